# baseline (speedup 1.0000x reference)
import functools

import jax
import jax.numpy as jnp
from jax import lax
from jax.experimental import pallas as pl
from jax.experimental.pallas import tpu as pltpu
from jax.experimental.pallas import tpu_sc as plsc

N = 10000
E = 320000
D = 128
NP = 10240
NC = 2
NS = 16
NW = NC * NS
EPW = E // NW
C = 80
NCH = EPW // C
EPP = NCH * C
NBC = 25
NBLK = NCH // NBC
STRIPE = NP // NS


@functools.lru_cache(maxsize=None)
def _sc_agg(W, with_deg):
    mesh = plsc.VectorSubcoreMesh(core_axis_name="c", subcore_axis_name="s",
                                  num_cores=NC, num_subcores=NS)
    NBUF = 16 if W <= 32 else (3 if with_deg else 4)
    out_type = [jax.ShapeDtypeStruct((NC, NP, W), jnp.float32)]
    scratch = [
        pltpu.VMEM((NBC, C), jnp.int32),
        pltpu.VMEM((NBC, C), jnp.int32),
        pltpu.VMEM((NBUF, C, W), jnp.float32),
        pltpu.VMEM_SHARED((NP, W), jnp.float32),
        pltpu.SemaphoreType.DMA,
        pltpu.SemaphoreType.DMA,
    ]
    if with_deg:
        out_type.append(jax.ShapeDtypeStruct((NC, NP), jnp.float32))
        scratch += [
            pltpu.VMEM((C,), jnp.float32),
            pltpu.VMEM_SHARED((NP,), jnp.float32),
            pltpu.SemaphoreType.DMA,
        ]

    def body(*refs):
        if with_deg:
            (table, src_r, dst_r, zrows,
             acc_out, deg_out, src_v, dst_v, rows_v, acc_sh, gsem, ssem,
             ones_v, deg_sh, dsem) = refs
        else:
            (table, src_r, dst_r, zrows,
             acc_out, src_v, dst_v, rows_v, acc_sh, gsem, ssem) = refs
        cid = lax.axis_index("c")
        sid = lax.axis_index("s")
        wid = sid * NC + cid
        base = sid * STRIPE
        for k in range(STRIPE // C):
            pltpu.sync_copy(zrows, acc_sh.at[pl.ds(base + k * C, C)])
        if with_deg:
            for j in range(C // 16):
                ones_v[pl.ds(j * 16, 16)] = jnp.zeros((16,), jnp.float32)
            for k in range(STRIPE // C):
                pltpu.sync_copy(ones_v, deg_sh.at[pl.ds(base + k * C, C)])
            for j in range(C // 16):
                ones_v[pl.ds(j * 16, 16)] = jnp.ones((16,), jnp.float32)
        plsc.subcore_barrier()

        def blk_body(blk, carry):
            pltpu.sync_copy(src_r.at[wid, blk], src_v)
            pltpu.sync_copy(dst_r.at[wid, blk], dst_v)
            for j in range(NBUF - 1):
                pltpu.async_copy(table.at[src_v.at[j]], rows_v.at[j], gsem)

            def step(i, carry2):
                b = lax.rem(i, NBUF)
                pltpu.make_async_copy(table.at[src_v.at[i]], rows_v.at[b],
                                      gsem).wait()
                pltpu.async_copy(rows_v.at[b], acc_sh.at[dst_v.at[i]],
                                 ssem, add=True)
                if with_deg:
                    pltpu.async_copy(ones_v, deg_sh.at[dst_v.at[i]],
                                     dsem, add=True)
                @pl.when(i > 0)
                def _():
                    pb = lax.rem(i + NBUF - 1, NBUF)
                    pltpu.make_async_copy(rows_v.at[pb],
                                          acc_sh.at[dst_v.at[i - 1]],
                                          ssem).wait()

                @pl.when(i + NBUF - 1 < NBC)
                def _():
                    pb = lax.rem(i + NBUF - 1, NBUF)
                    pltpu.async_copy(table.at[src_v.at[i + NBUF - 1]],
                                     rows_v.at[pb], gsem)
                return carry2

            lax.fori_loop(0, NBC, step, 0)
            lb = (NBC - 1) % NBUF
            pltpu.make_async_copy(rows_v.at[lb],
                                  acc_sh.at[dst_v.at[NBC - 1]],
                                  ssem).wait()
            if with_deg:
                def ddrain(k, carry3):
                    pltpu.make_async_copy(ones_v, deg_sh.at[dst_v.at[k]],
                                          dsem).wait()
                    return carry3

                lax.fori_loop(0, NBC, ddrain, 0)
            return carry

        lax.fori_loop(0, NBLK, blk_body, 0)
        plsc.subcore_barrier()
        pltpu.sync_copy(acc_sh.at[pl.ds(base, STRIPE)],
                        acc_out.at[cid, pl.ds(base, STRIPE)])
        if with_deg:
            pltpu.sync_copy(deg_sh.at[pl.ds(base, STRIPE)],
                            deg_out.at[cid, pl.ds(base, STRIPE)])

    params = None
    if W != D:
        params = pltpu.CompilerParams(use_tc_tiling_on_sc=False)
    return pl.kernel(body, out_type=out_type, mesh=mesh,
                     scratch_types=scratch, compiler_params=params)


R = 128
_FIXED = lambda b: (0, 0)
_ROWB = lambda b: (b, 0)
_ACCB = lambda b: (0, b, 0)


def _tc_self_call(WO):
    def kern(h_r, w_r, b_r, o_r):
        o_r[...] = lax.dot_general(h_r[...], w_r[...],
                                   (((1,), (1,)), ((), ())),
                                   preferred_element_type=jnp.float32) \
            + b_r[...]

    in_specs = [
        pl.BlockSpec((R, D), _ROWB),
        pl.BlockSpec((WO, D), _FIXED),
        pl.BlockSpec((1, WO), _FIXED),
    ]
    return pl.pallas_call(
        kern, grid=(NP // R,), in_specs=in_specs,
        out_specs=[pl.BlockSpec((R, WO), _ROWB)],
        out_shape=[jax.ShapeDtypeStruct((NP, WO), jnp.float32)])


def _tc_mid_call(proj):
    def kern(*refs):
        if proj:
            acc_r, deg_r, s_r, wl_r, g_r, be_r, w3_r, o_r, p_r = refs
        else:
            acc_r, deg_r, s_r, wl_r, g_r, be_r, o_r = refs
        a = acc_r[0] + acc_r[1]
        d = deg_r[0] + deg_r[1]
        mean = a * (1.0 / jnp.maximum(d, 1.0))
        z = lax.dot_general(mean, wl_r[...], (((1,), (1,)), ((), ())),
                            preferred_element_type=jnp.float32)
        z = z + s_r[...]
        h = jnp.maximum(z, 0.0)
        mu = jnp.mean(h, axis=1, keepdims=True)
        var = jnp.mean((h - mu) ** 2, axis=1, keepdims=True)
        out = (h - mu) * lax.rsqrt(var + 1e-5) * g_r[...] + be_r[...]
        o_r[...] = out
        if proj:
            p_r[...] = lax.dot_general(out, w3_r[...], (((1,), (1,)), ((), ())),
                                       preferred_element_type=jnp.float32)

    in_specs = [
        pl.BlockSpec((NC, R, D), _ACCB),
        pl.BlockSpec((NC, R, 1), _ACCB),
        pl.BlockSpec((R, D), _ROWB),
        pl.BlockSpec((D, D), _FIXED),
        pl.BlockSpec((1, D), _FIXED),
        pl.BlockSpec((1, D), _FIXED),
    ]
    out_shape = [jax.ShapeDtypeStruct((NP, D), jnp.float32)]
    out_specs = [pl.BlockSpec((R, D), _ROWB)]
    if proj:
        in_specs.append(pl.BlockSpec((16, D), _FIXED))
        out_shape.append(jax.ShapeDtypeStruct((NP, 16), jnp.float32))
        out_specs.append(pl.BlockSpec((R, 16), _ROWB))
    return pl.pallas_call(kern, grid=(NP // R,), in_specs=in_specs,
                          out_specs=out_specs, out_shape=out_shape)


def _tc_last_call():
    def kern(acc_r, deg_r, s_r, o_r):
        a = acc_r[0] + acc_r[1]
        d = deg_r[0] + deg_r[1]
        o_r[...] = a * (1.0 / jnp.maximum(d, 1.0)) + s_r[...]

    in_specs = [
        pl.BlockSpec((NC, R, 16), _ACCB),
        pl.BlockSpec((NC, R, 1), _ACCB),
        pl.BlockSpec((R, 16), _ROWB),
    ]
    return pl.pallas_call(
        kern, grid=(NP // R,), in_specs=in_specs,
        out_specs=[pl.BlockSpec((R, 16), _ROWB)],
        out_shape=[jax.ShapeDtypeStruct((NP, 16), jnp.float32)])


_self128 = _tc_self_call(D)
_self16 = _tc_self_call(16)
_layer1 = _tc_mid_call(False)
_layer2 = _tc_mid_call(True)
_layer3 = _tc_last_call()


def kernel(x, edge_index, W1l, b1, W1r, g1, be1, W2l, b2, W2r, g2, be2,
           W3l, b3, W3r):
    f32 = jnp.float32
    src = edge_index[0].astype(jnp.int32).reshape(NW, NBLK, NBC, C)
    dst = edge_index[1].astype(jnp.int32).reshape(NW, NBLK, NBC, C)
    x_pad = jnp.pad(x, ((0, NP - N), (0, 0)))
    z128 = jnp.zeros((C, D), f32)
    z16 = jnp.zeros((C, 16), f32)
    w3l_pad = jnp.pad(W3l, ((0, 16 - 3), (0, 0)))
    w3r_pad = jnp.pad(W3r, ((0, 16 - 3), (0, 0)))
    b3r = jnp.pad(b3, (0, 16 - 3)).reshape(1, 16)

    (s1,) = _self128(x_pad, W1r, b1.reshape(1, D))
    acc1, deg2 = _sc_agg(D, True)(x_pad, src, dst, z128)
    deg = deg2.reshape(NC, NP, 1)
    (h1,) = _layer1(acc1, deg, s1, W1l, g1.reshape(1, D), be1.reshape(1, D))
    (s2,) = _self128(h1, W2r, b2.reshape(1, D))
    (acc2,) = _sc_agg(D, False)(h1, src, dst, z128)
    h2, p3 = _layer2(acc2, deg, s2, W2l, g2.reshape(1, D), be2.reshape(1, D),
                     w3l_pad)
    (s3,) = _self16(h2, w3r_pad, b3r)
    (acc3,) = _sc_agg(16, False)(p3, src, dst, z16)
    (outp,) = _layer3(acc3, deg, s3)
    return outp[:N, :3]


# speedup vs baseline: 1.0274x; 1.0274x over previous
import functools

import jax
import jax.numpy as jnp
from jax import lax
from jax.experimental import pallas as pl
from jax.experimental.pallas import tpu as pltpu
from jax.experimental.pallas import tpu_sc as plsc

N = 10000
E = 320000
D = 128
NP = 10240
NC = 2
NS = 16
NW = NC * NS
EPW = E // NW
C = 80
NCH = EPW // C
EPP = NCH * C
NBC = 25
NBLK = NCH // NBC
STRIPE = NP // NS


@functools.lru_cache(maxsize=None)
def _sc_agg(W, with_deg):
    mesh = plsc.VectorSubcoreMesh(core_axis_name="c", subcore_axis_name="s",
                                  num_cores=NC, num_subcores=NS)
    NBUF = 16 if W <= 32 else (3 if with_deg else 4)
    out_type = [jax.ShapeDtypeStruct((NC, NP, W), jnp.float32)]
    scratch = [
        pltpu.VMEM((NBC, C), jnp.int32),
        pltpu.VMEM((NBC, C), jnp.int32),
        pltpu.VMEM((NBUF, C, W), jnp.float32),
        pltpu.VMEM_SHARED((NP, W), jnp.float32),
        pltpu.SemaphoreType.DMA,
        pltpu.SemaphoreType.DMA,
    ]
    if with_deg:
        out_type.append(jax.ShapeDtypeStruct((NC, NP), jnp.float32))
        scratch += [
            pltpu.VMEM((C,), jnp.float32),
            pltpu.VMEM_SHARED((NP,), jnp.float32),
            pltpu.SemaphoreType.DMA,
        ]

    def body(*refs):
        if with_deg:
            (table, src_r, dst_r, zrows,
             acc_out, deg_out, src_v, dst_v, rows_v, acc_sh, gsem, ssem,
             ones_v, deg_sh, dsem) = refs
        else:
            (table, src_r, dst_r, zrows,
             acc_out, src_v, dst_v, rows_v, acc_sh, gsem, ssem) = refs
        cid = lax.axis_index("c")
        sid = lax.axis_index("s")
        wid = sid * NC + cid
        base = sid * STRIPE
        for k in range(STRIPE // C):
            pltpu.sync_copy(zrows, acc_sh.at[pl.ds(base + k * C, C)])
        if with_deg:
            for j in range(C // 16):
                ones_v[pl.ds(j * 16, 16)] = jnp.zeros((16,), jnp.float32)
            for k in range(STRIPE // C):
                pltpu.sync_copy(ones_v, deg_sh.at[pl.ds(base + k * C, C)])
            for j in range(C // 16):
                ones_v[pl.ds(j * 16, 16)] = jnp.ones((16,), jnp.float32)
        plsc.subcore_barrier()

        def blk_body(blk, carry):
            pltpu.sync_copy(src_r.at[wid, blk], src_v)
            pltpu.sync_copy(dst_r.at[wid, blk], dst_v)
            for j in range(NBUF - 1):
                pltpu.async_copy(table.at[src_v.at[j]], rows_v.at[j], gsem)

            def step(i, carry2):
                b = lax.rem(i, NBUF)
                pltpu.make_async_copy(table.at[src_v.at[i]], rows_v.at[b],
                                      gsem).wait()
                pltpu.async_copy(rows_v.at[b], acc_sh.at[dst_v.at[i]],
                                 ssem, add=True)
                if with_deg:
                    pltpu.async_copy(ones_v, deg_sh.at[dst_v.at[i]],
                                     dsem, add=True)
                @pl.when(i > 0)
                def _():
                    pb = lax.rem(i + NBUF - 1, NBUF)
                    pltpu.make_async_copy(rows_v.at[pb],
                                          acc_sh.at[dst_v.at[i - 1]],
                                          ssem).wait()

                @pl.when(i + NBUF - 1 < NBC)
                def _():
                    pb = lax.rem(i + NBUF - 1, NBUF)
                    pltpu.async_copy(table.at[src_v.at[i + NBUF - 1]],
                                     rows_v.at[pb], gsem)
                return carry2

            lax.fori_loop(0, NBC, step, 0)
            lb = (NBC - 1) % NBUF
            pltpu.make_async_copy(rows_v.at[lb],
                                  acc_sh.at[dst_v.at[NBC - 1]],
                                  ssem).wait()
            if with_deg:
                def ddrain(k, carry3):
                    pltpu.make_async_copy(ones_v, deg_sh.at[dst_v.at[k]],
                                          dsem).wait()
                    return carry3

                lax.fori_loop(0, NBC, ddrain, 0)
            return carry

        lax.fori_loop(0, NBLK, blk_body, 0)
        plsc.subcore_barrier()
        pltpu.sync_copy(acc_sh.at[pl.ds(base, STRIPE)],
                        acc_out.at[cid, pl.ds(base, STRIPE)])
        if with_deg:
            pltpu.sync_copy(deg_sh.at[pl.ds(base, STRIPE)],
                            deg_out.at[cid, pl.ds(base, STRIPE)])

    params = None
    if W != D:
        params = pltpu.CompilerParams(use_tc_tiling_on_sc=False)
    return pl.kernel(body, out_type=out_type, mesh=mesh,
                     scratch_types=scratch, compiler_params=params)


R = 128
_FIXED = lambda b: (0, 0)
_ROWB = lambda b: (b, 0)
_ACCB = lambda b: (0, b, 0)


def _tc_mid_call(proj):
    def kern(*refs):
        if proj:
            (acc_r, deg_r, h_r, wl_r, wr_r, b_r, g_r, be_r, w3_r,
             o_r, p_r) = refs
        else:
            acc_r, deg_r, h_r, wl_r, wr_r, b_r, g_r, be_r, o_r = refs
        a = acc_r[0] + acc_r[1]
        d = deg_r[0] + deg_r[1]
        mean = a * (1.0 / jnp.maximum(d, 1.0))
        z = lax.dot_general(mean, wl_r[...], (((1,), (1,)), ((), ())),
                            preferred_element_type=jnp.float32)
        z = z + lax.dot_general(h_r[...], wr_r[...], (((1,), (1,)), ((), ())),
                                preferred_element_type=jnp.float32)
        z = z + b_r[...]
        h = jnp.maximum(z, 0.0)
        mu = jnp.mean(h, axis=1, keepdims=True)
        var = jnp.mean((h - mu) ** 2, axis=1, keepdims=True)
        out = (h - mu) * lax.rsqrt(var + 1e-5) * g_r[...] + be_r[...]
        o_r[...] = out
        if proj:
            p_r[...] = lax.dot_general(out, w3_r[...], (((1,), (1,)), ((), ())),
                                       preferred_element_type=jnp.float32)

    in_specs = [
        pl.BlockSpec((NC, R, D), _ACCB),
        pl.BlockSpec((NC, R, 1), _ACCB),
        pl.BlockSpec((R, D), _ROWB),
        pl.BlockSpec((D, D), _FIXED),
        pl.BlockSpec((D, D), _FIXED),
        pl.BlockSpec((1, D), _FIXED),
        pl.BlockSpec((1, D), _FIXED),
        pl.BlockSpec((1, D), _FIXED),
    ]
    out_shape = [jax.ShapeDtypeStruct((NP, D), jnp.float32)]
    out_specs = [pl.BlockSpec((R, D), _ROWB)]
    if proj:
        in_specs.append(pl.BlockSpec((16, D), _FIXED))
        out_shape.append(jax.ShapeDtypeStruct((NP, 16), jnp.float32))
        out_specs.append(pl.BlockSpec((R, 16), _ROWB))
    return pl.pallas_call(kern, grid=(NP // R,), in_specs=in_specs,
                          out_specs=out_specs, out_shape=out_shape)


def _tc_last_call():
    def kern(acc_r, deg_r, h_r, wr_r, b_r, o_r):
        a = acc_r[0] + acc_r[1]
        d = deg_r[0] + deg_r[1]
        z = a * (1.0 / jnp.maximum(d, 1.0))
        z = z + lax.dot_general(h_r[...], wr_r[...], (((1,), (1,)), ((), ())),
                                preferred_element_type=jnp.float32)
        o_r[...] = z + b_r[...]

    in_specs = [
        pl.BlockSpec((NC, R, 16), _ACCB),
        pl.BlockSpec((NC, R, 1), _ACCB),
        pl.BlockSpec((R, D), _ROWB),
        pl.BlockSpec((16, D), _FIXED),
        pl.BlockSpec((1, 16), _FIXED),
    ]
    return pl.pallas_call(
        kern, grid=(NP // R,), in_specs=in_specs,
        out_specs=[pl.BlockSpec((R, 16), _ROWB)],
        out_shape=[jax.ShapeDtypeStruct((NP, 16), jnp.float32)])


_layer1 = _tc_mid_call(False)
_layer2 = _tc_mid_call(True)
_layer3 = _tc_last_call()


def kernel(x, edge_index, W1l, b1, W1r, g1, be1, W2l, b2, W2r, g2, be2,
           W3l, b3, W3r):
    f32 = jnp.float32
    src = edge_index[0].astype(jnp.int32).reshape(NW, NBLK, NBC, C)
    dst = edge_index[1].astype(jnp.int32).reshape(NW, NBLK, NBC, C)
    x_pad = jnp.pad(x, ((0, NP - N), (0, 0)))
    z128 = jnp.zeros((C, D), f32)
    z16 = jnp.zeros((C, 16), f32)
    w3l_pad = jnp.pad(W3l, ((0, 16 - 3), (0, 0)))
    w3r_pad = jnp.pad(W3r, ((0, 16 - 3), (0, 0)))
    b3r = jnp.pad(b3, (0, 16 - 3)).reshape(1, 16)

    acc1, deg2 = _sc_agg(D, True)(x_pad, src, dst, z128)
    deg = deg2.reshape(NC, NP, 1)
    (h1,) = _layer1(acc1, deg, x_pad, W1l, W1r, b1.reshape(1, D),
                    g1.reshape(1, D), be1.reshape(1, D))
    (acc2,) = _sc_agg(D, False)(h1, src, dst, z128)
    h2, p3 = _layer2(acc2, deg, h1, W2l, W2r, b2.reshape(1, D),
                     g2.reshape(1, D), be2.reshape(1, D), w3l_pad)
    (acc3,) = _sc_agg(16, False)(p3, src, dst, z16)
    (outp,) = _layer3(acc3, deg, h2, w3r_pad, b3r)
    return outp[:N, :3]


# speedup vs baseline: 1.2023x; 1.1703x over previous
import functools

import jax
import jax.numpy as jnp
from jax import lax
from jax.experimental import pallas as pl
from jax.experimental.pallas import tpu as pltpu
from jax.experimental.pallas import tpu_sc as plsc

N = 10000
E = 320000
D = 128
NP = 10240
NC = 2
NS = 16
NW = NC * NS
EPW = E // NW
C = 80
NCH = EPW // C
EPP = NCH * C
NBC = 25
NBLK = NCH // NBC
STRIPE = NP // NS


@functools.lru_cache(maxsize=None)
def _sc_agg(W, with_deg):
    mesh = plsc.VectorSubcoreMesh(core_axis_name="c", subcore_axis_name="s",
                                  num_cores=NC, num_subcores=NS)
    NBUF = 16 if W <= 32 else (3 if with_deg else 4)
    out_type = [jax.ShapeDtypeStruct((NC, NP, W), jnp.float32)]
    scratch = [
        pltpu.VMEM((NBC, C), jnp.int32),
        pltpu.VMEM((NBC, C), jnp.int32),
        pltpu.VMEM((NBUF, C, W), jnp.float32),
        pltpu.VMEM_SHARED((NP, W), jnp.float32),
        pltpu.SemaphoreType.DMA,
        pltpu.SemaphoreType.DMA,
    ]
    if with_deg:
        out_type.append(jax.ShapeDtypeStruct((NC, NP), jnp.float32))
        scratch += [
            pltpu.VMEM((C,), jnp.float32),
            pltpu.VMEM_SHARED((NP,), jnp.float32),
            pltpu.SemaphoreType.DMA,
        ]

    def body(*refs):
        if with_deg:
            (table, src_r, dst_r, zrows,
             acc_out, deg_out, src_v, dst_v, rows_v, acc_sh, gsem, ssem,
             ones_v, deg_sh, dsem) = refs
        else:
            (table, src_r, dst_r, zrows,
             acc_out, src_v, dst_v, rows_v, acc_sh, gsem, ssem) = refs
        cid = lax.axis_index("c")
        sid = lax.axis_index("s")
        wid = sid * NC + cid
        base = sid * STRIPE
        for k in range(STRIPE // C):
            pltpu.sync_copy(zrows, acc_sh.at[pl.ds(base + k * C, C)])
        if with_deg:
            for j in range(C // 16):
                ones_v[pl.ds(j * 16, 16)] = jnp.zeros((16,), jnp.float32)
            for k in range(STRIPE // C):
                pltpu.sync_copy(ones_v, deg_sh.at[pl.ds(base + k * C, C)])
            for j in range(C // 16):
                ones_v[pl.ds(j * 16, 16)] = jnp.ones((16,), jnp.float32)
        plsc.subcore_barrier()

        def blk_body(blk, carry):
            pltpu.sync_copy(src_r.at[wid, blk], src_v)
            pltpu.sync_copy(dst_r.at[wid, blk], dst_v)
            for j in range(NBUF - 1):
                pltpu.async_copy(table.at[src_v.at[j]], rows_v.at[j], gsem)

            def step(i, carry2):
                b = lax.rem(i, NBUF)
                pltpu.make_async_copy(table.at[src_v.at[i]], rows_v.at[b],
                                      gsem).wait()
                pltpu.async_copy(rows_v.at[b], acc_sh.at[dst_v.at[i]],
                                 ssem, add=True)
                if with_deg:
                    pltpu.async_copy(ones_v, deg_sh.at[dst_v.at[i]],
                                     dsem, add=True)
                @pl.when(i > 0)
                def _():
                    pb = lax.rem(i + NBUF - 1, NBUF)
                    pltpu.make_async_copy(rows_v.at[pb],
                                          acc_sh.at[dst_v.at[i - 1]],
                                          ssem).wait()

                @pl.when(i + NBUF - 1 < NBC)
                def _():
                    pb = lax.rem(i + NBUF - 1, NBUF)
                    pltpu.async_copy(table.at[src_v.at[i + NBUF - 1]],
                                     rows_v.at[pb], gsem)
                return carry2

            lax.fori_loop(0, NBC, step, 0)
            lb = (NBC - 1) % NBUF
            pltpu.make_async_copy(rows_v.at[lb],
                                  acc_sh.at[dst_v.at[NBC - 1]],
                                  ssem).wait()
            if with_deg:
                def ddrain(k, carry3):
                    pltpu.make_async_copy(ones_v, deg_sh.at[dst_v.at[k]],
                                          dsem).wait()
                    return carry3

                lax.fori_loop(0, NBC, ddrain, 0)
            return carry

        lax.fori_loop(0, NBLK, blk_body, 0)
        plsc.subcore_barrier()
        pltpu.sync_copy(acc_sh.at[pl.ds(base, STRIPE)],
                        acc_out.at[cid, pl.ds(base, STRIPE)])
        if with_deg:
            pltpu.sync_copy(deg_sh.at[pl.ds(base, STRIPE)],
                            deg_out.at[cid, pl.ds(base, STRIPE)])

    params = None
    if W != D:
        params = pltpu.CompilerParams(use_tc_tiling_on_sc=False)
    return pl.kernel(body, out_type=out_type, mesh=mesh,
                     scratch_types=scratch, compiler_params=params)


R = 256
_FIXED = lambda b: (0, 0)
_ROWB = lambda b: (b, 0)
_ACCB = lambda b: (0, b, 0)


def _tc_mid_call(proj):
    def kern(*refs):
        if proj:
            (acc_r, deg_r, h_r, wl_r, wr_r, b_r, g_r, be_r, w3_r,
             o_r, p_r) = refs
        else:
            acc_r, deg_r, h_r, wl_r, wr_r, b_r, g_r, be_r, o_r = refs
        a = acc_r[0] + acc_r[1]
        d = deg_r[0] + deg_r[1]
        mean = a * (1.0 / jnp.maximum(d, 1.0))
        z = lax.dot_general(mean, wl_r[...], (((1,), (1,)), ((), ())),
                            preferred_element_type=jnp.float32)
        z = z + lax.dot_general(h_r[...], wr_r[...], (((1,), (1,)), ((), ())),
                                preferred_element_type=jnp.float32)
        z = z + b_r[...]
        h = jnp.maximum(z, 0.0)
        mu = jnp.mean(h, axis=1, keepdims=True)
        var = jnp.mean((h - mu) ** 2, axis=1, keepdims=True)
        out = (h - mu) * lax.rsqrt(var + 1e-5) * g_r[...] + be_r[...]
        o_r[...] = out
        if proj:
            p_r[...] = lax.dot_general(out, w3_r[...], (((1,), (1,)), ((), ())),
                                       preferred_element_type=jnp.float32)

    in_specs = [
        pl.BlockSpec((NC, R, D), _ACCB),
        pl.BlockSpec((NC, R, 1), _ACCB),
        pl.BlockSpec((R, D), _ROWB),
        pl.BlockSpec((D, D), _FIXED),
        pl.BlockSpec((D, D), _FIXED),
        pl.BlockSpec((1, D), _FIXED),
        pl.BlockSpec((1, D), _FIXED),
        pl.BlockSpec((1, D), _FIXED),
    ]
    out_shape = [jax.ShapeDtypeStruct((NP, D), jnp.float32)]
    out_specs = [pl.BlockSpec((R, D), _ROWB)]
    if proj:
        in_specs.append(pl.BlockSpec((16, D), _FIXED))
        out_shape.append(jax.ShapeDtypeStruct((NP, 16), jnp.float32))
        out_specs.append(pl.BlockSpec((R, 16), _ROWB))
    return pl.pallas_call(kern, grid=(NP // R,), in_specs=in_specs,
                          out_specs=out_specs, out_shape=out_shape)


def _tc_last_call():
    def kern(acc_r, deg_r, h_r, wr_r, b_r, o_r):
        a = acc_r[0] + acc_r[1]
        d = deg_r[0] + deg_r[1]
        z = a * (1.0 / jnp.maximum(d, 1.0))
        z = z + lax.dot_general(h_r[...], wr_r[...], (((1,), (1,)), ((), ())),
                                preferred_element_type=jnp.float32)
        o_r[...] = z + b_r[...]

    in_specs = [
        pl.BlockSpec((NC, R, 16), _ACCB),
        pl.BlockSpec((NC, R, 1), _ACCB),
        pl.BlockSpec((R, D), _ROWB),
        pl.BlockSpec((16, D), _FIXED),
        pl.BlockSpec((1, 16), _FIXED),
    ]
    return pl.pallas_call(
        kern, grid=(NP // R,), in_specs=in_specs,
        out_specs=[pl.BlockSpec((R, 16), _ROWB)],
        out_shape=[jax.ShapeDtypeStruct((NP, 16), jnp.float32)])


_layer1 = _tc_mid_call(False)
_layer2 = _tc_mid_call(True)
_layer3 = _tc_last_call()


def kernel(x, edge_index, W1l, b1, W1r, g1, be1, W2l, b2, W2r, g2, be2,
           W3l, b3, W3r):
    f32 = jnp.float32
    src = edge_index[0].astype(jnp.int32).reshape(NW, NBLK, NBC, C)
    dst = edge_index[1].astype(jnp.int32).reshape(NW, NBLK, NBC, C)
    x_pad = jnp.pad(x, ((0, NP - N), (0, 0)))
    z128 = jnp.zeros((C, D), f32)
    z16 = jnp.zeros((C, 16), f32)
    w3l_pad = jnp.pad(W3l, ((0, 16 - 3), (0, 0)))
    w3r_pad = jnp.pad(W3r, ((0, 16 - 3), (0, 0)))
    b3r = jnp.pad(b3, (0, 16 - 3)).reshape(1, 16)

    acc1, deg2 = _sc_agg(D, True)(x_pad, src, dst, z128)
    deg = deg2.reshape(NC, NP, 1)
    (h1,) = _layer1(acc1, deg, x_pad, W1l, W1r, b1.reshape(1, D),
                    g1.reshape(1, D), be1.reshape(1, D))
    (acc2,) = _sc_agg(D, False)(h1, src, dst, z128)
    h2, p3 = _layer2(acc2, deg, h1, W2l, W2r, b2.reshape(1, D),
                     g2.reshape(1, D), be2.reshape(1, D), w3l_pad)
    (acc3,) = _sc_agg(16, False)(p3, src, dst, z16)
    (outp,) = _layer3(acc3, deg, h2, w3r_pad, b3r)
    return outp[:N, :3]


# speedup vs baseline: 1.3120x; 1.0913x over previous
import functools

import jax
import jax.numpy as jnp
from jax import lax
from jax.experimental import pallas as pl
from jax.experimental.pallas import tpu as pltpu
from jax.experimental.pallas import tpu_sc as plsc

N = 10000
E = 320000
D = 128
NP = 10240
NC = 2
NS = 16
NW = NC * NS
EPW = E // NW
C = 80
NCH = EPW // C
EPP = NCH * C
NBC = 25
NBLK = NCH // NBC
STRIPE = NP // NS


@functools.lru_cache(maxsize=None)
def _sc_agg(W, with_deg):
    mesh = plsc.VectorSubcoreMesh(core_axis_name="c", subcore_axis_name="s",
                                  num_cores=NC, num_subcores=NS)
    NBUF = 16 if W <= 32 else (3 if with_deg else 4)
    out_type = [jax.ShapeDtypeStruct((NC, NP, W), jnp.float32)]
    scratch = [
        pltpu.VMEM((NBC, C), jnp.int32),
        pltpu.VMEM((NBC, C), jnp.int32),
        pltpu.VMEM((NBUF, C, W), jnp.float32),
        pltpu.VMEM_SHARED((NP, W), jnp.float32),
        pltpu.SemaphoreType.DMA,
        pltpu.SemaphoreType.DMA,
    ]
    if with_deg:
        out_type.append(jax.ShapeDtypeStruct((NC, NP), jnp.float32))
        scratch += [
            pltpu.VMEM((C,), jnp.float32),
            pltpu.VMEM_SHARED((NP,), jnp.float32),
            pltpu.SemaphoreType.DMA,
        ]

    def body(*refs):
        if with_deg:
            (table, src_r, dst_r, zrows,
             acc_out, deg_out, src_v, dst_v, rows_v, acc_sh, gsem, ssem,
             ones_v, deg_sh, dsem) = refs
        else:
            (table, src_r, dst_r, zrows,
             acc_out, src_v, dst_v, rows_v, acc_sh, gsem, ssem) = refs
        cid = lax.axis_index("c")
        sid = lax.axis_index("s")
        wid = sid * NC + cid
        base = sid * STRIPE
        for k in range(STRIPE // C):
            pltpu.sync_copy(zrows, acc_sh.at[pl.ds(base + k * C, C)])
        if with_deg:
            for j in range(C // 16):
                ones_v[pl.ds(j * 16, 16)] = jnp.zeros((16,), jnp.float32)
            for k in range(STRIPE // C):
                pltpu.sync_copy(ones_v, deg_sh.at[pl.ds(base + k * C, C)])
            for j in range(C // 16):
                ones_v[pl.ds(j * 16, 16)] = jnp.ones((16,), jnp.float32)
        plsc.subcore_barrier()

        def blk_body(blk, carry):
            pltpu.sync_copy(src_r.at[wid, blk], src_v)
            pltpu.sync_copy(dst_r.at[wid, blk], dst_v)
            for j in range(NBUF - 1):
                pltpu.async_copy(table.at[src_v.at[j]], rows_v.at[j], gsem)

            def step(i, carry2):
                b = lax.rem(i, NBUF)
                pltpu.make_async_copy(table.at[src_v.at[i]], rows_v.at[b],
                                      gsem).wait()
                pltpu.async_copy(rows_v.at[b], acc_sh.at[dst_v.at[i]],
                                 ssem, add=True)
                if with_deg:
                    pltpu.async_copy(ones_v, deg_sh.at[dst_v.at[i]],
                                     dsem, add=True)
                @pl.when(i > 0)
                def _():
                    pb = lax.rem(i + NBUF - 1, NBUF)
                    pltpu.make_async_copy(rows_v.at[pb],
                                          acc_sh.at[dst_v.at[i - 1]],
                                          ssem).wait()

                @pl.when(i + NBUF - 1 < NBC)
                def _():
                    pb = lax.rem(i + NBUF - 1, NBUF)
                    pltpu.async_copy(table.at[src_v.at[i + NBUF - 1]],
                                     rows_v.at[pb], gsem)
                return carry2

            lax.fori_loop(0, NBC, step, 0)
            lb = (NBC - 1) % NBUF
            pltpu.make_async_copy(rows_v.at[lb],
                                  acc_sh.at[dst_v.at[NBC - 1]],
                                  ssem).wait()
            if with_deg:
                def ddrain(k, carry3):
                    pltpu.make_async_copy(ones_v, deg_sh.at[dst_v.at[k]],
                                          dsem).wait()
                    return carry3

                lax.fori_loop(0, NBC, ddrain, 0)
            return carry

        lax.fori_loop(0, NBLK, blk_body, 0)
        plsc.subcore_barrier()
        pltpu.sync_copy(acc_sh.at[pl.ds(base, STRIPE)],
                        acc_out.at[cid, pl.ds(base, STRIPE)])
        if with_deg:
            pltpu.sync_copy(deg_sh.at[pl.ds(base, STRIPE)],
                            deg_out.at[cid, pl.ds(base, STRIPE)])

    params = None
    if W != D:
        params = pltpu.CompilerParams(use_tc_tiling_on_sc=False)
    return pl.kernel(body, out_type=out_type, mesh=mesh,
                     scratch_types=scratch, compiler_params=params)


R = 512
_FIXED = lambda b: (0, 0)
_ROWB = lambda b: (b, 0)
_ACCB = lambda b: (0, b, 0)


def _tc_mid_call(proj):
    def kern(*refs):
        if proj:
            (acc_r, deg_r, h_r, wl_r, wr_r, b_r, g_r, be_r, w3_r,
             o_r, p_r) = refs
        else:
            acc_r, deg_r, h_r, wl_r, wr_r, b_r, g_r, be_r, o_r = refs
        a = acc_r[0] + acc_r[1]
        d = deg_r[0] + deg_r[1]
        mean = a * (1.0 / jnp.maximum(d, 1.0))
        z = lax.dot_general(mean, wl_r[...], (((1,), (1,)), ((), ())),
                            preferred_element_type=jnp.float32)
        z = z + lax.dot_general(h_r[...], wr_r[...], (((1,), (1,)), ((), ())),
                                preferred_element_type=jnp.float32)
        z = z + b_r[...]
        h = jnp.maximum(z, 0.0)
        mu = jnp.mean(h, axis=1, keepdims=True)
        var = jnp.mean((h - mu) ** 2, axis=1, keepdims=True)
        out = (h - mu) * lax.rsqrt(var + 1e-5) * g_r[...] + be_r[...]
        o_r[...] = out
        if proj:
            p_r[...] = lax.dot_general(out, w3_r[...], (((1,), (1,)), ((), ())),
                                       preferred_element_type=jnp.float32)

    in_specs = [
        pl.BlockSpec((NC, R, D), _ACCB),
        pl.BlockSpec((NC, R, 1), _ACCB),
        pl.BlockSpec((R, D), _ROWB),
        pl.BlockSpec((D, D), _FIXED),
        pl.BlockSpec((D, D), _FIXED),
        pl.BlockSpec((1, D), _FIXED),
        pl.BlockSpec((1, D), _FIXED),
        pl.BlockSpec((1, D), _FIXED),
    ]
    out_shape = [jax.ShapeDtypeStruct((NP, D), jnp.float32)]
    out_specs = [pl.BlockSpec((R, D), _ROWB)]
    if proj:
        in_specs.append(pl.BlockSpec((16, D), _FIXED))
        out_shape.append(jax.ShapeDtypeStruct((NP, 16), jnp.float32))
        out_specs.append(pl.BlockSpec((R, 16), _ROWB))
    return pl.pallas_call(kern, grid=(NP // R,), in_specs=in_specs,
                          out_specs=out_specs, out_shape=out_shape)


def _tc_last_call():
    def kern(acc_r, deg_r, h_r, wr_r, b_r, o_r):
        a = acc_r[0] + acc_r[1]
        d = deg_r[0] + deg_r[1]
        z = a * (1.0 / jnp.maximum(d, 1.0))
        z = z + lax.dot_general(h_r[...], wr_r[...], (((1,), (1,)), ((), ())),
                                preferred_element_type=jnp.float32)
        o_r[...] = z + b_r[...]

    in_specs = [
        pl.BlockSpec((NC, R, 16), _ACCB),
        pl.BlockSpec((NC, R, 1), _ACCB),
        pl.BlockSpec((R, D), _ROWB),
        pl.BlockSpec((16, D), _FIXED),
        pl.BlockSpec((1, 16), _FIXED),
    ]
    return pl.pallas_call(
        kern, grid=(NP // R,), in_specs=in_specs,
        out_specs=[pl.BlockSpec((R, 16), _ROWB)],
        out_shape=[jax.ShapeDtypeStruct((NP, 16), jnp.float32)])


_layer1 = _tc_mid_call(False)
_layer2 = _tc_mid_call(True)
_layer3 = _tc_last_call()


def kernel(x, edge_index, W1l, b1, W1r, g1, be1, W2l, b2, W2r, g2, be2,
           W3l, b3, W3r):
    f32 = jnp.float32
    src = edge_index[0].astype(jnp.int32).reshape(NW, NBLK, NBC, C)
    dst = edge_index[1].astype(jnp.int32).reshape(NW, NBLK, NBC, C)
    x_pad = jnp.pad(x, ((0, NP - N), (0, 0)))
    z128 = jnp.zeros((C, D), f32)
    z16 = jnp.zeros((C, 16), f32)
    w3l_pad = jnp.pad(W3l, ((0, 16 - 3), (0, 0)))
    w3r_pad = jnp.pad(W3r, ((0, 16 - 3), (0, 0)))
    b3r = jnp.pad(b3, (0, 16 - 3)).reshape(1, 16)

    acc1, deg2 = _sc_agg(D, True)(x_pad, src, dst, z128)
    deg = deg2.reshape(NC, NP, 1)
    (h1,) = _layer1(acc1, deg, x_pad, W1l, W1r, b1.reshape(1, D),
                    g1.reshape(1, D), be1.reshape(1, D))
    (acc2,) = _sc_agg(D, False)(h1, src, dst, z128)
    h2, p3 = _layer2(acc2, deg, h1, W2l, W2r, b2.reshape(1, D),
                     g2.reshape(1, D), be2.reshape(1, D), w3l_pad)
    (acc3,) = _sc_agg(16, False)(p3, src, dst, z16)
    (outp,) = _layer3(acc3, deg, h2, w3r_pad, b3r)
    return outp[:N, :3]


# speedup vs baseline: 1.3737x; 1.0470x over previous
import functools

import jax
import jax.numpy as jnp
from jax import lax
from jax.experimental import pallas as pl
from jax.experimental.pallas import tpu as pltpu
from jax.experimental.pallas import tpu_sc as plsc

N = 10000
E = 320000
D = 128
NP = 10240
NC = 2
NS = 16
NW = NC * NS
EPW = E // NW
C = 80
NCH = EPW // C
EPP = NCH * C
NBC = 25
NBLK = NCH // NBC
STRIPE = NP // NS


@functools.lru_cache(maxsize=None)
def _sc_agg(W, with_deg):
    mesh = plsc.VectorSubcoreMesh(core_axis_name="c", subcore_axis_name="s",
                                  num_cores=NC, num_subcores=NS)
    NBUF = 16 if W <= 32 else (3 if with_deg else 4)
    out_type = [jax.ShapeDtypeStruct((NC, NP, W), jnp.float32)]
    scratch = [
        pltpu.VMEM((NBC, C), jnp.int32),
        pltpu.VMEM((NBC, C), jnp.int32),
        pltpu.VMEM((NBUF, C, W), jnp.float32),
        pltpu.VMEM_SHARED((NP, W), jnp.float32),
        pltpu.SemaphoreType.DMA,
        pltpu.SemaphoreType.DMA,
    ]
    if with_deg:
        out_type.append(jax.ShapeDtypeStruct((NC, NP), jnp.float32))
        scratch += [
            pltpu.VMEM((C,), jnp.float32),
            pltpu.VMEM_SHARED((NP,), jnp.float32),
            pltpu.SemaphoreType.DMA,
        ]

    def body(*refs):
        if with_deg:
            (table, src_r, dst_r, zrows,
             acc_out, deg_out, src_v, dst_v, rows_v, acc_sh, gsem, ssem,
             ones_v, deg_sh, dsem) = refs
        else:
            (table, src_r, dst_r, zrows,
             acc_out, src_v, dst_v, rows_v, acc_sh, gsem, ssem) = refs
        cid = lax.axis_index("c")
        sid = lax.axis_index("s")
        wid = sid * NC + cid
        base = sid * STRIPE
        for k in range(STRIPE // C):
            pltpu.sync_copy(zrows, acc_sh.at[pl.ds(base + k * C, C)])
        if with_deg:
            for j in range(C // 16):
                ones_v[pl.ds(j * 16, 16)] = jnp.zeros((16,), jnp.float32)
            for k in range(STRIPE // C):
                pltpu.sync_copy(ones_v, deg_sh.at[pl.ds(base + k * C, C)])
            for j in range(C // 16):
                ones_v[pl.ds(j * 16, 16)] = jnp.ones((16,), jnp.float32)
        plsc.subcore_barrier()

        def blk_body(blk, carry):
            pltpu.sync_copy(src_r.at[wid, blk], src_v)
            pltpu.sync_copy(dst_r.at[wid, blk], dst_v)
            for j in range(NBUF - 1):
                pltpu.async_copy(table.at[src_v.at[j]], rows_v.at[j], gsem)

            def step(i, carry2):
                b = lax.rem(i, NBUF)
                pltpu.make_async_copy(table.at[src_v.at[i]], rows_v.at[b],
                                      gsem).wait()
                pltpu.async_copy(rows_v.at[b], acc_sh.at[dst_v.at[i]],
                                 ssem, add=True)
                if with_deg:
                    pltpu.async_copy(ones_v, deg_sh.at[dst_v.at[i]],
                                     dsem, add=True)
                @pl.when(i > 0)
                def _():
                    pb = lax.rem(i + NBUF - 1, NBUF)
                    pltpu.make_async_copy(rows_v.at[pb],
                                          acc_sh.at[dst_v.at[i - 1]],
                                          ssem).wait()

                @pl.when(i + NBUF - 1 < NBC)
                def _():
                    pb = lax.rem(i + NBUF - 1, NBUF)
                    pltpu.async_copy(table.at[src_v.at[i + NBUF - 1]],
                                     rows_v.at[pb], gsem)
                return carry2

            lax.fori_loop(0, NBC, step, 0)
            lb = (NBC - 1) % NBUF
            pltpu.make_async_copy(rows_v.at[lb],
                                  acc_sh.at[dst_v.at[NBC - 1]],
                                  ssem).wait()
            if with_deg:
                def ddrain(k, carry3):
                    pltpu.make_async_copy(ones_v, deg_sh.at[dst_v.at[k]],
                                          dsem).wait()
                    return carry3

                lax.fori_loop(0, NBC, ddrain, 0)
            return carry

        lax.fori_loop(0, NBLK, blk_body, 0)
        plsc.subcore_barrier()
        pltpu.sync_copy(acc_sh.at[pl.ds(base, STRIPE)],
                        acc_out.at[cid, pl.ds(base, STRIPE)])
        if with_deg:
            pltpu.sync_copy(deg_sh.at[pl.ds(base, STRIPE)],
                            deg_out.at[cid, pl.ds(base, STRIPE)])

    params = None
    if W != D:
        params = pltpu.CompilerParams(use_tc_tiling_on_sc=False)
    return pl.kernel(body, out_type=out_type, mesh=mesh,
                     scratch_types=scratch, compiler_params=params)


R = 1024
_FIXED = lambda b: (0, 0)
_ROWB = lambda b: (b, 0)
_ACCB = lambda b: (0, b, 0)


def _tc_mid_call(proj):
    def kern(*refs):
        if proj:
            (acc_r, deg_r, h_r, wl_r, wr_r, b_r, g_r, be_r, w3_r,
             o_r, p_r) = refs
        else:
            acc_r, deg_r, h_r, wl_r, wr_r, b_r, g_r, be_r, o_r = refs
        a = acc_r[0] + acc_r[1]
        d = deg_r[0] + deg_r[1]
        mean = a * (1.0 / jnp.maximum(d, 1.0))
        z = lax.dot_general(mean, wl_r[...], (((1,), (1,)), ((), ())),
                            preferred_element_type=jnp.float32)
        z = z + lax.dot_general(h_r[...], wr_r[...], (((1,), (1,)), ((), ())),
                                preferred_element_type=jnp.float32)
        z = z + b_r[...]
        h = jnp.maximum(z, 0.0)
        mu = jnp.mean(h, axis=1, keepdims=True)
        var = jnp.mean((h - mu) ** 2, axis=1, keepdims=True)
        out = (h - mu) * lax.rsqrt(var + 1e-5) * g_r[...] + be_r[...]
        o_r[...] = out
        if proj:
            p_r[...] = lax.dot_general(out, w3_r[...], (((1,), (1,)), ((), ())),
                                       preferred_element_type=jnp.float32)

    in_specs = [
        pl.BlockSpec((NC, R, D), _ACCB),
        pl.BlockSpec((NC, R, 1), _ACCB),
        pl.BlockSpec((R, D), _ROWB),
        pl.BlockSpec((D, D), _FIXED),
        pl.BlockSpec((D, D), _FIXED),
        pl.BlockSpec((1, D), _FIXED),
        pl.BlockSpec((1, D), _FIXED),
        pl.BlockSpec((1, D), _FIXED),
    ]
    out_shape = [jax.ShapeDtypeStruct((NP, D), jnp.float32)]
    out_specs = [pl.BlockSpec((R, D), _ROWB)]
    if proj:
        in_specs.append(pl.BlockSpec((16, D), _FIXED))
        out_shape.append(jax.ShapeDtypeStruct((NP, 16), jnp.float32))
        out_specs.append(pl.BlockSpec((R, 16), _ROWB))
    return pl.pallas_call(kern, grid=(NP // R,), in_specs=in_specs,
                          out_specs=out_specs, out_shape=out_shape)


def _tc_last_call():
    def kern(acc_r, deg_r, h_r, wr_r, b_r, o_r):
        a = acc_r[0] + acc_r[1]
        d = deg_r[0] + deg_r[1]
        z = a * (1.0 / jnp.maximum(d, 1.0))
        z = z + lax.dot_general(h_r[...], wr_r[...], (((1,), (1,)), ((), ())),
                                preferred_element_type=jnp.float32)
        o_r[...] = z + b_r[...]

    in_specs = [
        pl.BlockSpec((NC, R, 16), _ACCB),
        pl.BlockSpec((NC, R, 1), _ACCB),
        pl.BlockSpec((R, D), _ROWB),
        pl.BlockSpec((16, D), _FIXED),
        pl.BlockSpec((1, 16), _FIXED),
    ]
    return pl.pallas_call(
        kern, grid=(NP // R,), in_specs=in_specs,
        out_specs=[pl.BlockSpec((R, 16), _ROWB)],
        out_shape=[jax.ShapeDtypeStruct((NP, 16), jnp.float32)])


_layer1 = _tc_mid_call(False)
_layer2 = _tc_mid_call(True)
_layer3 = _tc_last_call()


def kernel(x, edge_index, W1l, b1, W1r, g1, be1, W2l, b2, W2r, g2, be2,
           W3l, b3, W3r):
    f32 = jnp.float32
    src = edge_index[0].astype(jnp.int32).reshape(NW, NBLK, NBC, C)
    dst = edge_index[1].astype(jnp.int32).reshape(NW, NBLK, NBC, C)
    x_pad = jnp.pad(x, ((0, NP - N), (0, 0)))
    z128 = jnp.zeros((C, D), f32)
    z16 = jnp.zeros((C, 16), f32)
    w3l_pad = jnp.pad(W3l, ((0, 16 - 3), (0, 0)))
    w3r_pad = jnp.pad(W3r, ((0, 16 - 3), (0, 0)))
    b3r = jnp.pad(b3, (0, 16 - 3)).reshape(1, 16)

    acc1, deg2 = _sc_agg(D, True)(x_pad, src, dst, z128)
    deg = deg2.reshape(NC, NP, 1)
    (h1,) = _layer1(acc1, deg, x_pad, W1l, W1r, b1.reshape(1, D),
                    g1.reshape(1, D), be1.reshape(1, D))
    (acc2,) = _sc_agg(D, False)(h1, src, dst, z128)
    h2, p3 = _layer2(acc2, deg, h1, W2l, W2r, b2.reshape(1, D),
                     g2.reshape(1, D), be2.reshape(1, D), w3l_pad)
    (acc3,) = _sc_agg(16, False)(p3, src, dst, z16)
    (outp,) = _layer3(acc3, deg, h2, w3r_pad, b3r)
    return outp[:N, :3]


# speedup vs baseline: 1.3949x; 1.0154x over previous
import functools

import jax
import jax.numpy as jnp
from jax import lax
from jax.experimental import pallas as pl
from jax.experimental.pallas import tpu as pltpu
from jax.experimental.pallas import tpu_sc as plsc

N = 10000
E = 320000
D = 128
NP = 10240
NC = 2
NS = 16
NW = NC * NS
EPW = E // NW
C = 80
NCH = EPW // C
EPP = NCH * C
NBC = 25
NBLK = NCH // NBC
STRIPE = NP // NS


@functools.lru_cache(maxsize=None)
def _sc_agg(W, with_deg):
    mesh = plsc.VectorSubcoreMesh(core_axis_name="c", subcore_axis_name="s",
                                  num_cores=NC, num_subcores=NS)
    NBUF = 16 if W <= 32 else (3 if with_deg else 4)
    out_type = [jax.ShapeDtypeStruct((NC, NP, W), jnp.float32)]
    scratch = [
        pltpu.VMEM((NBC, C), jnp.int32),
        pltpu.VMEM((NBC, C), jnp.int32),
        pltpu.VMEM((NBUF, C, W), jnp.float32),
        pltpu.VMEM_SHARED((NP, W), jnp.float32),
        pltpu.SemaphoreType.DMA,
        pltpu.SemaphoreType.DMA,
    ]
    if with_deg:
        out_type.append(jax.ShapeDtypeStruct((NC, NP), jnp.float32))
        scratch += [
            pltpu.VMEM((C,), jnp.float32),
            pltpu.VMEM_SHARED((NP,), jnp.float32),
            pltpu.SemaphoreType.DMA,
        ]

    def body(*refs):
        if with_deg:
            (table, src_r, dst_r, zrows,
             acc_out, deg_out, src_v, dst_v, rows_v, acc_sh, gsem, ssem,
             ones_v, deg_sh, dsem) = refs
        else:
            (table, src_r, dst_r, zrows,
             acc_out, src_v, dst_v, rows_v, acc_sh, gsem, ssem) = refs
        cid = lax.axis_index("c")
        sid = lax.axis_index("s")
        wid = sid * NC + cid
        base = sid * STRIPE
        for k in range(STRIPE // C):
            pltpu.sync_copy(zrows, acc_sh.at[pl.ds(base + k * C, C)])
        if with_deg:
            for j in range(C // 16):
                ones_v[pl.ds(j * 16, 16)] = jnp.zeros((16,), jnp.float32)
            for k in range(STRIPE // C):
                pltpu.sync_copy(ones_v, deg_sh.at[pl.ds(base + k * C, C)])
            for j in range(C // 16):
                ones_v[pl.ds(j * 16, 16)] = jnp.ones((16,), jnp.float32)
        plsc.subcore_barrier()

        def blk_body(blk, carry):
            pltpu.sync_copy(src_r.at[wid, blk], src_v)
            pltpu.sync_copy(dst_r.at[wid, blk], dst_v)
            for j in range(NBUF - 1):
                pltpu.async_copy(table.at[src_v.at[j]], rows_v.at[j], gsem)

            def step(i, carry2):
                b = lax.rem(i, NBUF)
                pltpu.make_async_copy(table.at[src_v.at[i]], rows_v.at[b],
                                      gsem).wait()
                pltpu.async_copy(rows_v.at[b], acc_sh.at[dst_v.at[i]],
                                 ssem, add=True)
                if with_deg:
                    pltpu.async_copy(ones_v, deg_sh.at[dst_v.at[i]],
                                     dsem, add=True)
                @pl.when(i > 0)
                def _():
                    pb = lax.rem(i + NBUF - 1, NBUF)
                    pltpu.make_async_copy(rows_v.at[pb],
                                          acc_sh.at[dst_v.at[i - 1]],
                                          ssem).wait()

                @pl.when(i + NBUF - 1 < NBC)
                def _():
                    pb = lax.rem(i + NBUF - 1, NBUF)
                    pltpu.async_copy(table.at[src_v.at[i + NBUF - 1]],
                                     rows_v.at[pb], gsem)
                return carry2

            lax.fori_loop(0, NBC, step, 0)
            lb = (NBC - 1) % NBUF
            pltpu.make_async_copy(rows_v.at[lb],
                                  acc_sh.at[dst_v.at[NBC - 1]],
                                  ssem).wait()
            if with_deg:
                def ddrain(k, carry3):
                    pltpu.make_async_copy(ones_v, deg_sh.at[dst_v.at[k]],
                                          dsem).wait()
                    return carry3

                lax.fori_loop(0, NBC, ddrain, 0)
            return carry

        lax.fori_loop(0, NBLK, blk_body, 0)
        plsc.subcore_barrier()
        pltpu.sync_copy(acc_sh.at[pl.ds(base, STRIPE)],
                        acc_out.at[cid, pl.ds(base, STRIPE)])
        if with_deg:
            pltpu.sync_copy(deg_sh.at[pl.ds(base, STRIPE)],
                            deg_out.at[cid, pl.ds(base, STRIPE)])

    params = None
    if W != D:
        params = pltpu.CompilerParams(use_tc_tiling_on_sc=False)
    return pl.kernel(body, out_type=out_type, mesh=mesh,
                     scratch_types=scratch, compiler_params=params)


R = 2048
_FIXED = lambda b: (0, 0)
_ROWB = lambda b: (b, 0)
_ACCB = lambda b: (0, b, 0)


def _tc_mid_call(proj):
    def kern(*refs):
        if proj:
            (acc_r, deg_r, h_r, wl_r, wr_r, b_r, g_r, be_r, w3_r,
             o_r, p_r) = refs
        else:
            acc_r, deg_r, h_r, wl_r, wr_r, b_r, g_r, be_r, o_r = refs
        a = acc_r[0] + acc_r[1]
        d = deg_r[0] + deg_r[1]
        mean = a * (1.0 / jnp.maximum(d, 1.0))
        z = lax.dot_general(mean, wl_r[...], (((1,), (1,)), ((), ())),
                            preferred_element_type=jnp.float32)
        z = z + lax.dot_general(h_r[...], wr_r[...], (((1,), (1,)), ((), ())),
                                preferred_element_type=jnp.float32)
        z = z + b_r[...]
        h = jnp.maximum(z, 0.0)
        mu = jnp.mean(h, axis=1, keepdims=True)
        var = jnp.mean((h - mu) ** 2, axis=1, keepdims=True)
        out = (h - mu) * lax.rsqrt(var + 1e-5) * g_r[...] + be_r[...]
        o_r[...] = out
        if proj:
            p_r[...] = lax.dot_general(out, w3_r[...], (((1,), (1,)), ((), ())),
                                       preferred_element_type=jnp.float32)

    in_specs = [
        pl.BlockSpec((NC, R, D), _ACCB),
        pl.BlockSpec((NC, R, 1), _ACCB),
        pl.BlockSpec((R, D), _ROWB),
        pl.BlockSpec((D, D), _FIXED),
        pl.BlockSpec((D, D), _FIXED),
        pl.BlockSpec((1, D), _FIXED),
        pl.BlockSpec((1, D), _FIXED),
        pl.BlockSpec((1, D), _FIXED),
    ]
    out_shape = [jax.ShapeDtypeStruct((NP, D), jnp.float32)]
    out_specs = [pl.BlockSpec((R, D), _ROWB)]
    if proj:
        in_specs.append(pl.BlockSpec((16, D), _FIXED))
        out_shape.append(jax.ShapeDtypeStruct((NP, 16), jnp.float32))
        out_specs.append(pl.BlockSpec((R, 16), _ROWB))
    return pl.pallas_call(kern, grid=(NP // R,), in_specs=in_specs,
                          out_specs=out_specs, out_shape=out_shape)


def _tc_last_call():
    def kern(acc_r, deg_r, h_r, wr_r, b_r, o_r):
        a = acc_r[0] + acc_r[1]
        d = deg_r[0] + deg_r[1]
        z = a * (1.0 / jnp.maximum(d, 1.0))
        z = z + lax.dot_general(h_r[...], wr_r[...], (((1,), (1,)), ((), ())),
                                preferred_element_type=jnp.float32)
        o_r[...] = z + b_r[...]

    in_specs = [
        pl.BlockSpec((NC, R, 16), _ACCB),
        pl.BlockSpec((NC, R, 1), _ACCB),
        pl.BlockSpec((R, D), _ROWB),
        pl.BlockSpec((16, D), _FIXED),
        pl.BlockSpec((1, 16), _FIXED),
    ]
    return pl.pallas_call(
        kern, grid=(NP // R,), in_specs=in_specs,
        out_specs=[pl.BlockSpec((R, 16), _ROWB)],
        out_shape=[jax.ShapeDtypeStruct((NP, 16), jnp.float32)])


_layer1 = _tc_mid_call(False)
_layer2 = _tc_mid_call(True)
_layer3 = _tc_last_call()


def kernel(x, edge_index, W1l, b1, W1r, g1, be1, W2l, b2, W2r, g2, be2,
           W3l, b3, W3r):
    f32 = jnp.float32
    src = edge_index[0].astype(jnp.int32).reshape(NW, NBLK, NBC, C)
    dst = edge_index[1].astype(jnp.int32).reshape(NW, NBLK, NBC, C)
    x_pad = jnp.pad(x, ((0, NP - N), (0, 0)))
    z128 = jnp.zeros((C, D), f32)
    z16 = jnp.zeros((C, 16), f32)
    w3l_pad = jnp.pad(W3l, ((0, 16 - 3), (0, 0)))
    w3r_pad = jnp.pad(W3r, ((0, 16 - 3), (0, 0)))
    b3r = jnp.pad(b3, (0, 16 - 3)).reshape(1, 16)

    acc1, deg2 = _sc_agg(D, True)(x_pad, src, dst, z128)
    deg = deg2.reshape(NC, NP, 1)
    (h1,) = _layer1(acc1, deg, x_pad, W1l, W1r, b1.reshape(1, D),
                    g1.reshape(1, D), be1.reshape(1, D))
    (acc2,) = _sc_agg(D, False)(h1, src, dst, z128)
    h2, p3 = _layer2(acc2, deg, h1, W2l, W2r, b2.reshape(1, D),
                     g2.reshape(1, D), be2.reshape(1, D), w3l_pad)
    (acc3,) = _sc_agg(16, False)(p3, src, dst, z16)
    (outp,) = _layer3(acc3, deg, h2, w3r_pad, b3r)
    return outp[:N, :3]
